# baseline (speedup 1.0000x reference)
import functools

import jax
import jax.numpy as jnp
from jax import lax
from jax.experimental import pallas as pl
from jax.experimental.pallas import tpu as pltpu
from jax.experimental.pallas import tpu_sc as plsc

B = 16384
D = 64

NC = 2
NS = 16
NW = NC * NS
BPW = B // NW
CH = 128
NCH = BPW // CH

@functools.cache
def _build_sc_gather4():
    mesh = plsc.VectorSubcoreMesh(
        core_axis_name="c", subcore_axis_name="s",
        num_cores=NC, num_subcores=NS,
    )
    return pl.kernel(
        _sc_gather4_body,
        out_type=[jax.ShapeDtypeStruct((B, D), jnp.float32)] * 4,
        mesh=mesh,
        scratch_types=[
            pltpu.VMEM((NCH, CH), jnp.int32),
            pltpu.VMEM((NCH, CH), jnp.int32),
            pltpu.VMEM((BPW, D), jnp.float32),
            pltpu.VMEM((BPW, D), jnp.float32),
            pltpu.SemaphoreType.DMA,
            pltpu.SemaphoreType.DMA,
        ],
        compiler_params=pltpu.CompilerParams(use_tc_tiling_on_sc=False),
    )


def _sc_gather4_body(users, items, ue_mlp, ie_mlp, ue_gmf, ie_gmf,
                     o_um, o_im, o_ug, o_ig,
                     idx_u, idx_i, buf_a, buf_b, sem_a, sem_b):
    wid = lax.axis_index("s") * NC + lax.axis_index("c")
    base = wid * BPW

    for j in range(NCH):
        pltpu.sync_copy(users.at[pl.ds(base + j * CH, CH)], idx_u.at[j])
        pltpu.sync_copy(items.at[pl.ds(base + j * CH, CH)], idx_i.at[j])

    def fire(table, idx, buf, sem):
        return [
            pltpu.async_copy(table.at[idx.at[j]],
                             buf.at[pl.ds(j * CH, CH)], sem)
            for j in range(NCH)
        ]

    def drain(handles):
        for h in handles:
            h.wait()

    h0 = fire(ue_mlp, idx_u, buf_a, sem_a)
    h1 = fire(ie_mlp, idx_i, buf_b, sem_b)
    drain(h0)
    pltpu.sync_copy(buf_a, o_um.at[pl.ds(base, BPW)])
    h2 = fire(ue_gmf, idx_u, buf_a, sem_a)
    drain(h1)
    pltpu.sync_copy(buf_b, o_im.at[pl.ds(base, BPW)])
    h3 = fire(ie_gmf, idx_i, buf_b, sem_b)
    drain(h2)
    pltpu.sync_copy(buf_a, o_ug.at[pl.ds(base, BPW)])
    drain(h3)
    pltpu.sync_copy(buf_b, o_ig.at[pl.ds(base, BPW)])


BLK = 2048


def _mlp_body(um, im, ug, ig, w1u, w1i, b1, w2, b2, w3, b3, wg, bg, wf, bf,
              out):
    dot = functools.partial(
        jnp.dot,
        precision=lax.Precision.HIGHEST,
        preferred_element_type=jnp.float32,
    )
    h = jnp.maximum(dot(um[...], w1u[...]) + dot(im[...], w1i[...]) + b1[...],
                    0.0)
    h = jnp.maximum(dot(h, w2[...]) + b2[...], 0.0)
    h = dot(h, w3[...]) + b3[...]
    g = dot(ug[...] * ig[...], wg[...]) + bg[...]
    out[...] = dot(h, wf[...][:, :32].T) + dot(g, wf[...][:, 32:].T) + bf[...]


def _tc_head(um, im, ug, ig, w1u, w1i, b1, w2, b2, w3, b3, wg, bg, wf, bf):
    row_spec = pl.BlockSpec((BLK, D), lambda i: (i, 0))
    full = lambda a: pl.BlockSpec(a.shape, lambda i: (0,) * a.ndim)
    return pl.pallas_call(
        _mlp_body,
        grid=(B // BLK,),
        in_specs=[row_spec] * 4 + [
            full(w1u), full(w1i), full(b1), full(w2), full(b2), full(w3),
            full(b3), full(wg), full(bg), full(wf), full(bf),
        ],
        out_specs=pl.BlockSpec((BLK, 1), lambda i: (i, 0)),
        out_shape=jax.ShapeDtypeStruct((B, 1), jnp.float32),
    )(um, im, ug, ig, w1u, w1i, b1, w2, b2, w3, b3, wg, bg, wf, bf)


@jax.jit
def kernel(users, items, ue_mlp, ie_mlp, ue_gmf, ie_gmf, W_gmf, b_gmf,
           W1, b1, W2, b2, W3, b3, Wf, bf):
    um, im, ug, ig = _build_sc_gather4()(
        users, items, ue_mlp, ie_mlp, ue_gmf, ie_gmf)
    out = _tc_head(
        um, im, ug, ig,
        W1[:, :D].T, W1[:, D:].T, b1[None, :],
        W2.T, b2[None, :],
        W3.T, b3[None, :],
        W_gmf.T, b_gmf[None, :],
        Wf, bf[None, :],
    )
    return jnp.squeeze(out, axis=1)
